# baseline (speedup 1.0000x reference)
import functools

import jax
import jax.numpy as jnp
from jax import lax
from jax.experimental import pallas as pl
from jax.experimental.pallas import tpu as pltpu
from jax.experimental.pallas import tpu_sc as plsc

D = 4
N_NODES = 10000
N_HEDGES = 10000
NNZ = 320000
F_DIM = 128
TW = 16
OUT = NNZ * 16

NW = 32
PER_W = NNZ // NW
CH = 400
NCH = PER_W // CH
NG = CH // 16


def _tables_body(x_ref, e_ref, wx_ref, we_ref, crow_ref, gx_ref, ge_ref):
    def one(src_ref, w_ref, dst_ref, extra):
        v = src_ref[...]
        bn = v.shape[0] // D
        m = v.reshape(bn, D, F_DIM).mean(axis=1)
        r = jnp.dot(m, w_ref[...], preferred_element_type=jnp.float32,
                    precision=lax.Precision.HIGHEST)
        q = jnp.sum(m * m, axis=1, keepdims=True)
        s = jnp.sum(m, axis=1, keepdims=True)
        colid = lax.broadcasted_iota(jnp.int32, r.shape, 1)
        out = jnp.where(colid == 7, q, jnp.where(colid == 6, s, r))
        if extra is not None:
            out = out + extra
        dst_ref[...] = out

    one(x_ref, wx_ref, gx_ref, None)
    one(e_ref, we_ref, ge_ref, crow_ref[...])


def _make_tables(x, e, wxp, wep, crow):
    BN = 400
    grid = N_NODES // BN
    return pl.pallas_call(
        _tables_body,
        grid=(grid,),
        in_specs=[
            pl.BlockSpec((D * BN, F_DIM), lambda i: (i, 0)),
            pl.BlockSpec((D * BN, F_DIM), lambda i: (i, 0)),
            pl.BlockSpec((F_DIM, TW), lambda i: (0, 0)),
            pl.BlockSpec((F_DIM, TW), lambda i: (0, 0)),
            pl.BlockSpec((1, TW), lambda i: (0, 0)),
        ],
        out_specs=[
            pl.BlockSpec((BN, TW), lambda i: (i, 0)),
            pl.BlockSpec((BN, TW), lambda i: (i, 0)),
        ],
        out_shape=[
            jax.ShapeDtypeStruct((N_NODES, TW), jnp.float32),
            jax.ShapeDtypeStruct((N_HEDGES, TW), jnp.float32),
        ],
    )(x, e, wxp, wep, crow)


def _mul(a, b):
    if isinstance(a, float):
        if a == 0.0:
            return 0.0
        if a == 1.0:
            return b
    if isinstance(b, float):
        if b == 0.0:
            return 0.0
        if b == 1.0:
            return a
    return a * b


def _add(a, b):
    if isinstance(a, float) and a == 0.0:
        return b
    if isinstance(b, float) and b == 0.0:
        return a
    return a + b


def _sub(a, b):
    if isinstance(b, float) and b == 0.0:
        return a
    return a - b


def _householder_q(p, rcp=None):
    if rcp is None:
        rcp = lambda a: 1.0 / a
    v = [
        [1.0, p[0], p[1], p[3]],
        [0.0, 1.0, p[2], p[4]],
        [0.0, 0.0, 1.0, p[5]],
        [0.0, 0.0, 0.0, 1.0],
    ]
    tau = [
        2.0 * rcp(1.0 + p[0] * p[0] + p[1] * p[1] + p[3] * p[3]),
        2.0 * rcp(1.0 + p[2] * p[2] + p[4] * p[4]),
        2.0 * rcp(1.0 + p[5] * p[5]),
        2.0,
    ]
    Q = [[1.0 if i == j else 0.0 for j in range(4)] for i in range(4)]
    for i in (3, 2, 1, 0):
        w = [0.0, 0.0, 0.0, 0.0]
        for j in range(4):
            acc = 0.0
            for k in range(4):
                acc = _add(acc, _mul(v[i][k], Q[k][j]))
            w[j] = acc
        for k in range(4):
            tv = _mul(tau[i], v[i][k])
            for j in range(4):
                Q[k][j] = _sub(Q[k][j], _mul(tv, w[j]))
    out = []
    for i in range(4):
        for j in range(4):
            q = Q[i][j]
            if isinstance(q, float):
                q = jnp.full_like(p[0], q)
            out.append(q)
    return out


def _rcp16(a):
    i = plsc.bitcast(a, jnp.int32)
    y = plsc.bitcast(0x7EF311C3 - i, jnp.float32)
    for _ in range(3):
        y = y * (2.0 - a * y)
    return y


def _tanh16(z):
    z = jnp.minimum(jnp.maximum(z, -20.0), 20.0)
    return 1.0 - 2.0 * _rcp16(jnp.exp(z + z) + 1.0)


def _rsqrt16(x):
    i = plsc.bitcast(x, jnp.int32)
    i = 0x5F3759DF - lax.shift_right_arithmetic(i, 1)
    y = plsc.bitcast(i, jnp.float32)
    for _ in range(3):
        y = y * (1.5 - 0.5 * x * y * y)
    return y


def _sc_body(gx_hbm, ge_hbm, row_hbm, col_hbm,
             hidx_hbm, attr_hbm,
             ridx_v, cidx_v, g1_v, g2_v,
             attr_v, h0_v, h1_v, sem1, sem2):
    wid = lax.axis_index("s") * 2 + lax.axis_index("c")
    iota = lax.iota(jnp.int32, 16)

    def chunk(ch, carry):
        base = wid * PER_W + ch * CH
        pltpu.sync_copy(row_hbm.at[pl.ds(base, CH)], ridx_v)
        pltpu.sync_copy(col_hbm.at[pl.ds(base, CH)], cidx_v)
        a = pltpu.async_copy(gx_hbm.at[ridx_v], g1_v, sem1)
        c = pltpu.async_copy(ge_hbm.at[cidx_v], g2_v, sem2)
        a.wait()
        c.wait()

        def group(g, carry2):
            b16 = g * 16
            pi = b16 + iota
            row16 = ridx_v[pl.ds(b16, 16)]
            col16 = cidx_v[pl.ds(b16, 16)]
            f1 = [plsc.load_gather(g1_v, [pi, jnp.full((16,), t, jnp.int32)])
                  for t in range(8)]
            f2 = [plsc.load_gather(g2_v, [pi, jnp.full((16,), t, jnp.int32)])
                  for t in range(14)]
            inv = 1.0 / (2 * F_DIM)
            mu = (f1[6] + f2[6]) * inv
            m2 = (f1[7] + f2[7]) * inv
            var = m2 - mu * mu
            rs = _rsqrt16(var + 1e-5)
            p = [_tanh16((f1[t] + f2[t]) * rs + f2[8 + t])
                 for t in range(6)]
            q = _householder_q(p, rcp=_rcp16)
            oidx = pi * 16
            for k in range(16):
                plsc.store_scatter(attr_v, [oidx + k], q[k])
            r4 = row16 * D
            c4 = col16 * D
            for k in range(16):
                plsc.store_scatter(h0_v, [oidx + k], r4 + (k // D))
                plsc.store_scatter(h1_v, [oidx + k], c4 + (k % D))
            return carry2

        lax.fori_loop(0, NG, group, 0)
        ob = base * 16
        pltpu.sync_copy(attr_v, attr_hbm.at[pl.ds(ob, CH * 16)])
        pltpu.sync_copy(h0_v, hidx_hbm.at[pl.ds(ob, CH * 16)])
        pltpu.sync_copy(h1_v, hidx_hbm.at[pl.ds(OUT + ob, CH * 16)])
        return carry

    lax.fori_loop(0, NCH, chunk, 0)


def _sc_epilogue(gx, ge, row, col):
    run = pl.kernel(
        _sc_body,
        out_type=[
            jax.ShapeDtypeStruct((2 * OUT,), jnp.int32),
            jax.ShapeDtypeStruct((OUT,), jnp.float32),
        ],
        mesh=plsc.VectorSubcoreMesh(core_axis_name="c", subcore_axis_name="s"),
        compiler_params=pltpu.CompilerParams(
            use_tc_tiling_on_sc=False, needs_layout_passes=False),
        scratch_types=[
            pltpu.VMEM((CH,), jnp.int32),
            pltpu.VMEM((CH,), jnp.int32),
            pltpu.VMEM((CH, TW), jnp.float32),
            pltpu.VMEM((CH, TW), jnp.float32),
            pltpu.VMEM((CH * 16,), jnp.float32),
            pltpu.VMEM((CH * 16,), jnp.int32),
            pltpu.VMEM((CH * 16,), jnp.int32),
            pltpu.SemaphoreType.DMA,
            pltpu.SemaphoreType.DMA,
        ],
    )
    return run(gx, ge, row, col)


def kernel(x, e, hyperedge_index, ln_w, ln_b, W, b):
    f = F_DIM
    wf = ln_w[:, None] * W
    sw = wf.sum(0)
    wx6 = wf[:f] - sw[None, :] * (1.0 / (2 * f))
    we6 = wf[f:] - sw[None, :] * (1.0 / (2 * f))
    pad = jnp.zeros((f, TW - 7), jnp.float32)
    ones = jnp.ones((f, 1), jnp.float32)
    wxp = jnp.concatenate([wx6, ones, pad], axis=1)
    wep = jnp.concatenate([we6, ones, pad], axis=1)
    c0 = ln_b @ W + b
    crow = jnp.concatenate([
        jnp.zeros((8,), jnp.float32), c0, jnp.zeros((2,), jnp.float32),
    ]).reshape(1, TW)

    row = hyperedge_index[0].astype(jnp.int32)
    col = hyperedge_index[1].astype(jnp.int32)

    gx, ge = _make_tables(x, e, wxp, wep, crow)
    hflat, attr = _sc_epilogue(gx, ge, row, col)
    return (hflat.reshape(2, OUT), attr)


# speedup vs baseline: 2.0193x; 2.0193x over previous
import functools

import jax
import jax.numpy as jnp
from jax import lax
from jax.experimental import pallas as pl
from jax.experimental.pallas import tpu as pltpu
from jax.experimental.pallas import tpu_sc as plsc

D = 4
N_NODES = 10000
N_HEDGES = 10000
NNZ = 320000
F_DIM = 128
TW = 16
OUT = NNZ * 16

NW = 32
PER_W = NNZ // NW
CH = 400
NCH = PER_W // CH
NG = CH // 16


def _tables_body(x_ref, e_ref, wx_ref, we_ref, crow_ref, gx_ref, ge_ref):
    def one(src_ref, w_ref, dst_ref, extra):
        v = src_ref[...]
        bn = v.shape[0] // D
        m = v.reshape(bn, D, F_DIM).mean(axis=1)
        r = jnp.dot(m, w_ref[...], preferred_element_type=jnp.float32,
                    precision=lax.Precision.HIGHEST)
        q = jnp.sum(m * m, axis=1, keepdims=True)
        s = jnp.sum(m, axis=1, keepdims=True)
        colid = lax.broadcasted_iota(jnp.int32, r.shape, 1)
        out = jnp.where(colid == 7, q, jnp.where(colid == 6, s, r))
        if extra is not None:
            out = out + extra
        dst_ref[...] = out

    one(x_ref, wx_ref, gx_ref, None)
    one(e_ref, we_ref, ge_ref, crow_ref[...])


def _make_tables(x, e, wxp, wep, crow):
    BN = 400
    grid = N_NODES // BN
    return pl.pallas_call(
        _tables_body,
        grid=(grid,),
        in_specs=[
            pl.BlockSpec((D * BN, F_DIM), lambda i: (i, 0)),
            pl.BlockSpec((D * BN, F_DIM), lambda i: (i, 0)),
            pl.BlockSpec((F_DIM, TW), lambda i: (0, 0)),
            pl.BlockSpec((F_DIM, TW), lambda i: (0, 0)),
            pl.BlockSpec((1, TW), lambda i: (0, 0)),
        ],
        out_specs=[
            pl.BlockSpec((BN, TW), lambda i: (i, 0)),
            pl.BlockSpec((BN, TW), lambda i: (i, 0)),
        ],
        out_shape=[
            jax.ShapeDtypeStruct((N_NODES, TW), jnp.float32),
            jax.ShapeDtypeStruct((N_HEDGES, TW), jnp.float32),
        ],
    )(x, e, wxp, wep, crow)


def _mul(a, b):
    if isinstance(a, float):
        if a == 0.0:
            return 0.0
        if a == 1.0:
            return b
    if isinstance(b, float):
        if b == 0.0:
            return 0.0
        if b == 1.0:
            return a
    return a * b


def _add(a, b):
    if isinstance(a, float) and a == 0.0:
        return b
    if isinstance(b, float) and b == 0.0:
        return a
    return a + b


def _sub(a, b):
    if isinstance(b, float) and b == 0.0:
        return a
    return a - b


def _householder_q(p, rcp=None):
    if rcp is None:
        rcp = lambda a: 1.0 / a
    v = [
        [1.0, p[0], p[1], p[3]],
        [0.0, 1.0, p[2], p[4]],
        [0.0, 0.0, 1.0, p[5]],
        [0.0, 0.0, 0.0, 1.0],
    ]
    tau = [
        2.0 * rcp(1.0 + p[0] * p[0] + p[1] * p[1] + p[3] * p[3]),
        2.0 * rcp(1.0 + p[2] * p[2] + p[4] * p[4]),
        2.0 * rcp(1.0 + p[5] * p[5]),
        2.0,
    ]
    Q = [[1.0 if i == j else 0.0 for j in range(4)] for i in range(4)]
    for i in (3, 2, 1, 0):
        w = [0.0, 0.0, 0.0, 0.0]
        for j in range(4):
            acc = 0.0
            for k in range(4):
                acc = _add(acc, _mul(v[i][k], Q[k][j]))
            w[j] = acc
        for k in range(4):
            tv = _mul(tau[i], v[i][k])
            for j in range(4):
                Q[k][j] = _sub(Q[k][j], _mul(tv, w[j]))
    out = []
    for i in range(4):
        for j in range(4):
            q = Q[i][j]
            if isinstance(q, float):
                q = jnp.full_like(p[0], q)
            out.append(q)
    return out


def _rcp16(a):
    i = plsc.bitcast(a, jnp.int32)
    y = plsc.bitcast(0x7EF311C3 - i, jnp.float32)
    for _ in range(3):
        y = y * (2.0 - a * y)
    return y


def _tanh16(z):
    z = jnp.minimum(jnp.maximum(z, -20.0), 20.0)
    return 1.0 - 2.0 * _rcp16(jnp.exp(z + z) + 1.0)


def _rsqrt16(x):
    i = plsc.bitcast(x, jnp.int32)
    i = 0x5F3759DF - lax.shift_right_arithmetic(i, 1)
    y = plsc.bitcast(i, jnp.float32)
    for _ in range(3):
        y = y * (1.5 - 0.5 * x * y * y)
    return y


def _sc_body(gx_hbm, ge_hbm, row_hbm, col_hbm,
             hidx_hbm, attr_hbm,
             ridx_v, cidx_v, g1_v, g2_v,
             attr_v, h0_v, h1_v, sem1, sem2):
    wid = lax.axis_index("s") * 2 + lax.axis_index("c")
    iota = lax.iota(jnp.int32, 16)

    def chunk(ch, carry):
        base = wid * PER_W + ch * CH
        pltpu.sync_copy(row_hbm.at[pl.ds(base, CH)], ridx_v)
        pltpu.sync_copy(col_hbm.at[pl.ds(base, CH)], cidx_v)
        a = pltpu.async_copy(gx_hbm.at[ridx_v], g1_v, sem1)
        c = pltpu.async_copy(ge_hbm.at[cidx_v], g2_v, sem2)
        a.wait()
        c.wait()

        def group(g, carry2):
            b16 = g * 16
            pi = b16 + iota
            row16 = ridx_v[pl.ds(b16, 16)]
            col16 = cidx_v[pl.ds(b16, 16)]
            f1 = [plsc.load_gather(g1_v, [pi, jnp.full((16,), t, jnp.int32)])
                  for t in range(8)]
            f2 = [plsc.load_gather(g2_v, [pi, jnp.full((16,), t, jnp.int32)])
                  for t in range(14)]
            inv = 1.0 / (2 * F_DIM)
            mu = (f1[6] + f2[6]) * inv
            m2 = (f1[7] + f2[7]) * inv
            var = m2 - mu * mu
            rs = _rsqrt16(var + 1e-5)
            p = [_tanh16((f1[t] + f2[t]) * rs + f2[8 + t])
                 for t in range(6)]
            q = _householder_q(p, rcp=_rcp16)
            oidx = pi * 16
            for k in range(16):
                plsc.store_scatter(attr_v, [oidx + k], q[k])
            r4 = row16 * D
            c4 = col16 * D
            for k in range(16):
                plsc.store_scatter(h0_v, [oidx + k], r4 + (k // D))
                plsc.store_scatter(h1_v, [oidx + k], c4 + (k % D))
            return carry2

        lax.fori_loop(0, NG, group, 0)
        ob = base * 16
        pltpu.sync_copy(attr_v, attr_hbm.at[pl.ds(ob, CH * 16)])
        pltpu.sync_copy(h0_v, hidx_hbm.at[pl.ds(ob, CH * 16)])
        pltpu.sync_copy(h1_v, hidx_hbm.at[pl.ds(OUT + ob, CH * 16)])
        return carry

    lax.fori_loop(0, NCH, chunk, 0)


def _sc_epilogue(gx, ge, row, col):
    run = pl.kernel(
        _sc_body,
        out_type=[
            jax.ShapeDtypeStruct((2 * OUT,), jnp.int32),
            jax.ShapeDtypeStruct((OUT,), jnp.float32),
        ],
        mesh=plsc.VectorSubcoreMesh(core_axis_name="c", subcore_axis_name="s"),
        compiler_params=pltpu.CompilerParams(
            use_tc_tiling_on_sc=False, needs_layout_passes=False),
        scratch_types=[
            pltpu.VMEM((CH,), jnp.int32),
            pltpu.VMEM((CH,), jnp.int32),
            pltpu.VMEM((CH, TW), jnp.float32),
            pltpu.VMEM((CH, TW), jnp.float32),
            pltpu.VMEM((CH * 16,), jnp.float32),
            pltpu.VMEM((CH * 16,), jnp.int32),
            pltpu.VMEM((CH * 16,), jnp.int32),
            pltpu.SemaphoreType.DMA,
            pltpu.SemaphoreType.DMA,
        ],
    )
    return run(gx, ge, row, col)


def _copy_body(a_ref, b_ref, ao_ref, bo_ref):
    ao_ref[...] = a_ref[...]
    bo_ref[...] = b_ref[...]


def _copy_out(hflat, attr):
    g = 100
    hw = (2 * OUT) // (g * 8)
    aw = OUT // (g * 8)
    hv = hflat.reshape(g, 8, hw)
    av = attr.reshape(g, 8, aw)
    return pl.pallas_call(
        _copy_body,
        grid=(g,),
        in_specs=[
            pl.BlockSpec((1, 8, hw), lambda i: (i, 0, 0)),
            pl.BlockSpec((1, 8, aw), lambda i: (i, 0, 0)),
        ],
        out_specs=[
            pl.BlockSpec((1, 8, hw), lambda i: (i, 0, 0)),
            pl.BlockSpec((1, 8, aw), lambda i: (i, 0, 0)),
        ],
        out_shape=[
            jax.ShapeDtypeStruct((g, 8, hw), jnp.int32),
            jax.ShapeDtypeStruct((g, 8, aw), jnp.float32),
        ],
    )(hv, av)


def kernel(x, e, hyperedge_index, ln_w, ln_b, W, b):
    f = F_DIM
    wf = ln_w[:, None] * W
    sw = wf.sum(0)
    wx6 = wf[:f] - sw[None, :] * (1.0 / (2 * f))
    we6 = wf[f:] - sw[None, :] * (1.0 / (2 * f))
    pad = jnp.zeros((f, TW - 7), jnp.float32)
    ones = jnp.ones((f, 1), jnp.float32)
    wxp = jnp.concatenate([wx6, ones, pad], axis=1)
    wep = jnp.concatenate([we6, ones, pad], axis=1)
    c0 = ln_b @ W + b
    crow = jnp.concatenate([
        jnp.zeros((8,), jnp.float32), c0, jnp.zeros((2,), jnp.float32),
    ]).reshape(1, TW)

    row = hyperedge_index[0].astype(jnp.int32)
    col = hyperedge_index[1].astype(jnp.int32)

    gx, ge = _make_tables(x, e, wxp, wep, crow)
    hflat, attr = _sc_epilogue(gx, ge, row, col)
    hc, ac = _copy_out(hflat, attr)
    return (hc.reshape(2, OUT), ac.reshape(OUT))


# speedup vs baseline: 2.3238x; 1.1508x over previous
import functools

import jax
import jax.numpy as jnp
from jax import lax
from jax.experimental import pallas as pl
from jax.experimental.pallas import tpu as pltpu
from jax.experimental.pallas import tpu_sc as plsc

D = 4
N_NODES = 10000
N_HEDGES = 10000
NNZ = 320000
F_DIM = 128
TW = 16
OUT = NNZ * 16

NW = 32
PER_W = NNZ // NW
CH = 400
NCH = PER_W // CH
NG = CH // 16


def _tables_body(x_ref, e_ref, wx_ref, we_ref, crow_ref, gx_ref, ge_ref):
    def one(src_ref, w_ref, dst_ref, extra):
        v = src_ref[...]
        bn = v.shape[0] // D
        m = v.reshape(bn, D, F_DIM).mean(axis=1)
        r = jnp.dot(m, w_ref[...], preferred_element_type=jnp.float32,
                    precision=lax.Precision.HIGHEST)
        q = jnp.sum(m * m, axis=1, keepdims=True)
        s = jnp.sum(m, axis=1, keepdims=True)
        colid = lax.broadcasted_iota(jnp.int32, r.shape, 1)
        out = jnp.where(colid == 7, q, jnp.where(colid == 6, s, r))
        if extra is not None:
            out = out + extra
        dst_ref[...] = out

    one(x_ref, wx_ref, gx_ref, None)
    one(e_ref, we_ref, ge_ref, crow_ref[...])


def _make_tables(x, e, wxp, wep, crow):
    BN = 400
    grid = N_NODES // BN
    return pl.pallas_call(
        _tables_body,
        grid=(grid,),
        in_specs=[
            pl.BlockSpec((D * BN, F_DIM), lambda i: (i, 0)),
            pl.BlockSpec((D * BN, F_DIM), lambda i: (i, 0)),
            pl.BlockSpec((F_DIM, TW), lambda i: (0, 0)),
            pl.BlockSpec((F_DIM, TW), lambda i: (0, 0)),
            pl.BlockSpec((1, TW), lambda i: (0, 0)),
        ],
        out_specs=[
            pl.BlockSpec((BN, TW), lambda i: (i, 0)),
            pl.BlockSpec((BN, TW), lambda i: (i, 0)),
        ],
        out_shape=[
            jax.ShapeDtypeStruct((N_NODES, TW), jnp.float32),
            jax.ShapeDtypeStruct((N_HEDGES, TW), jnp.float32),
        ],
    )(x, e, wxp, wep, crow)


def _mul(a, b):
    if isinstance(a, float):
        if a == 0.0:
            return 0.0
        if a == 1.0:
            return b
    if isinstance(b, float):
        if b == 0.0:
            return 0.0
        if b == 1.0:
            return a
    return a * b


def _add(a, b):
    if isinstance(a, float) and a == 0.0:
        return b
    if isinstance(b, float) and b == 0.0:
        return a
    return a + b


def _sub(a, b):
    if isinstance(b, float) and b == 0.0:
        return a
    return a - b


def _householder_q(p, rcp=None):
    if rcp is None:
        rcp = lambda a: 1.0 / a
    v = [
        [1.0, p[0], p[1], p[3]],
        [0.0, 1.0, p[2], p[4]],
        [0.0, 0.0, 1.0, p[5]],
        [0.0, 0.0, 0.0, 1.0],
    ]
    tau = [
        2.0 * rcp(1.0 + p[0] * p[0] + p[1] * p[1] + p[3] * p[3]),
        2.0 * rcp(1.0 + p[2] * p[2] + p[4] * p[4]),
        2.0 * rcp(1.0 + p[5] * p[5]),
        2.0,
    ]
    Q = [[1.0 if i == j else 0.0 for j in range(4)] for i in range(4)]
    for i in (3, 2, 1, 0):
        w = [0.0, 0.0, 0.0, 0.0]
        for j in range(4):
            acc = 0.0
            for k in range(4):
                acc = _add(acc, _mul(v[i][k], Q[k][j]))
            w[j] = acc
        for k in range(4):
            tv = _mul(tau[i], v[i][k])
            for j in range(4):
                Q[k][j] = _sub(Q[k][j], _mul(tv, w[j]))
    out = []
    for i in range(4):
        for j in range(4):
            q = Q[i][j]
            if isinstance(q, float):
                q = jnp.full_like(p[0], q)
            out.append(q)
    return out


def _rcp16(a):
    i = plsc.bitcast(a, jnp.int32)
    y = plsc.bitcast(0x7EF311C3 - i, jnp.float32)
    for _ in range(3):
        y = y * (2.0 - a * y)
    return y


def _tanh16(z):
    z = jnp.minimum(jnp.maximum(z, -20.0), 20.0)
    return 1.0 - 2.0 * _rcp16(jnp.exp(z + z) + 1.0)


def _rsqrt16(x):
    i = plsc.bitcast(x, jnp.int32)
    i = 0x5F3759DF - lax.shift_right_arithmetic(i, 1)
    y = plsc.bitcast(i, jnp.float32)
    for _ in range(3):
        y = y * (1.5 - 0.5 * x * y * y)
    return y


def _sc_body(gx_hbm, ge_hbm, row_hbm, col_hbm,
             h0_hbm, h1_hbm, attr_hbm,
             ridx_v, cidx_v, g1_v, g2_v,
             attr_v, h0_v, h1_v, sem1, sem2):
    wid = lax.axis_index("s") * 2 + lax.axis_index("c")
    iota = lax.iota(jnp.int32, 16)

    def chunk(ch, carry):
        base = wid * PER_W + ch * CH
        pltpu.sync_copy(row_hbm.at[pl.ds(base, CH)], ridx_v)
        pltpu.sync_copy(col_hbm.at[pl.ds(base, CH)], cidx_v)
        a = pltpu.async_copy(gx_hbm.at[ridx_v], g1_v, sem1)
        c = pltpu.async_copy(ge_hbm.at[cidx_v], g2_v, sem2)
        a.wait()
        c.wait()

        def group(g, carry2):
            b16 = g * 16
            pi = b16 + iota
            row16 = ridx_v[pl.ds(b16, 16)]
            col16 = cidx_v[pl.ds(b16, 16)]
            f1 = [plsc.load_gather(g1_v, [pi, jnp.full((16,), t, jnp.int32)])
                  for t in range(8)]
            f2 = [plsc.load_gather(g2_v, [pi, jnp.full((16,), t, jnp.int32)])
                  for t in range(14)]
            inv = 1.0 / (2 * F_DIM)
            mu = (f1[6] + f2[6]) * inv
            m2 = (f1[7] + f2[7]) * inv
            var = m2 - mu * mu
            rs = _rsqrt16(var + 1e-5)
            p = [_tanh16((f1[t] + f2[t]) * rs + f2[8 + t])
                 for t in range(6)]
            q = _householder_q(p, rcp=_rcp16)
            oidx = pi * 16
            for k in range(16):
                plsc.store_scatter(attr_v, [oidx + k], q[k])
            r4 = row16 * D
            c4 = col16 * D
            for k in range(16):
                plsc.store_scatter(h0_v, [oidx + k], r4 + (k // D))
                plsc.store_scatter(h1_v, [oidx + k], c4 + (k % D))
            return carry2

        lax.fori_loop(0, NG, group, 0)
        ob = base * 16
        pltpu.sync_copy(attr_v, attr_hbm.at[pl.ds(ob, CH * 16)])
        pltpu.sync_copy(h0_v, h0_hbm.at[pl.ds(ob, CH * 16)])
        pltpu.sync_copy(h1_v, h1_hbm.at[pl.ds(ob, CH * 16)])
        return carry

    lax.fori_loop(0, NCH, chunk, 0)


def _sc_epilogue(gx, ge, row, col):
    run = pl.kernel(
        _sc_body,
        out_type=[
            jax.ShapeDtypeStruct((OUT,), jnp.int32),
            jax.ShapeDtypeStruct((OUT,), jnp.int32),
            jax.ShapeDtypeStruct((OUT,), jnp.float32),
        ],
        mesh=plsc.VectorSubcoreMesh(core_axis_name="c", subcore_axis_name="s"),
        compiler_params=pltpu.CompilerParams(
            use_tc_tiling_on_sc=False, needs_layout_passes=False),
        scratch_types=[
            pltpu.VMEM((CH,), jnp.int32),
            pltpu.VMEM((CH,), jnp.int32),
            pltpu.VMEM((CH, TW), jnp.float32),
            pltpu.VMEM((CH, TW), jnp.float32),
            pltpu.VMEM((CH * 16,), jnp.float32),
            pltpu.VMEM((CH * 16,), jnp.int32),
            pltpu.VMEM((CH * 16,), jnp.int32),
            pltpu.SemaphoreType.DMA,
            pltpu.SemaphoreType.DMA,
        ],
    )
    return run(gx, ge, row, col)


def _finish_body(h0_ref, h1_ref, a_ref, hout_ref, aout_ref):
    hout_ref[...] = jnp.concatenate(
        [h0_ref[...][None, :], h1_ref[...][None, :]], axis=0)
    aout_ref[...] = a_ref[...]


def _finish(h0p, h1p, attr):
    BL = 20480
    g = OUT // BL
    return pl.pallas_call(
        _finish_body,
        grid=(g,),
        in_specs=[
            pl.BlockSpec((BL,), lambda i: (i,)),
            pl.BlockSpec((BL,), lambda i: (i,)),
            pl.BlockSpec((BL,), lambda i: (i,)),
        ],
        out_specs=[
            pl.BlockSpec((2, BL), lambda i: (0, i)),
            pl.BlockSpec((BL,), lambda i: (i,)),
        ],
        out_shape=[
            jax.ShapeDtypeStruct((2, OUT), jnp.int32),
            jax.ShapeDtypeStruct((OUT,), jnp.float32),
        ],
    )(h0p, h1p, attr)


def kernel(x, e, hyperedge_index, ln_w, ln_b, W, b):
    f = F_DIM
    wf = ln_w[:, None] * W
    sw = wf.sum(0)
    wx6 = wf[:f] - sw[None, :] * (1.0 / (2 * f))
    we6 = wf[f:] - sw[None, :] * (1.0 / (2 * f))
    pad = jnp.zeros((f, TW - 7), jnp.float32)
    ones = jnp.ones((f, 1), jnp.float32)
    wxp = jnp.concatenate([wx6, ones, pad], axis=1)
    wep = jnp.concatenate([we6, ones, pad], axis=1)
    c0 = ln_b @ W + b
    crow = jnp.concatenate([
        jnp.zeros((8,), jnp.float32), c0, jnp.zeros((2,), jnp.float32),
    ]).reshape(1, TW)

    row = hyperedge_index[0].astype(jnp.int32)
    col = hyperedge_index[1].astype(jnp.int32)

    gx, ge = _make_tables(x, e, wxp, wep, crow)
    h0p, h1p, attr = _sc_epilogue(gx, ge, row, col)
    return _finish(h0p, h1p, attr)
